# baseline (speedup 1.0000x reference)
import functools

import jax
import jax.numpy as jnp
from jax import lax
from jax.experimental import pallas as pl
from jax.experimental.pallas import tpu as pltpu
from jax.experimental.pallas import tpu_sc as plsc

NUM_ENTITIES = 1000000
NUM_RELATIONS = 1000
EMBED_DIM = 64
BATCH = 16384

NC = 2
NS = 16
NW = NC * NS
BPW = BATCH // NW
LANES = 16
NGROUP = BPW // LANES


def _make_sc_lookup():
  mesh = plsc.VectorSubcoreMesh(core_axis_name="c", subcore_axis_name="s")

  @functools.partial(
      pl.kernel,
      mesh=mesh,
      compiler_params=pltpu.CompilerParams(needs_layout_passes=False),
      out_type=(
          jax.ShapeDtypeStruct((BATCH, EMBED_DIM), jnp.float32),
          jax.ShapeDtypeStruct((BATCH, EMBED_DIM), jnp.float32),
      ),
      scratch_types=[
          pltpu.VMEM((BPW,), jnp.int32),
          pltpu.VMEM((BPW,), jnp.int32),
          pltpu.SemaphoreType.DMA,
          pltpu.SemaphoreType.DMA,
      ],
  )
  def lookup(heads_hbm, rels_hbm, ent_hbm, rel_hbm, out_h, out_r,
             hidx_v, ridx_v, hsem, rsem):
    wid = lax.axis_index("s") * NC + lax.axis_index("c")
    base = wid * BPW
    pltpu.sync_copy(heads_hbm.at[wid], hidx_v)
    pltpu.sync_copy(rels_hbm.at[wid], ridx_v)

    def fire(g, _):
      gbase = g * LANES
      hvec = hidx_v[pl.ds(gbase, LANES)]
      rvec = ridx_v[pl.ds(gbase, LANES)]
      for j in range(LANES):
        pltpu.async_copy(ent_hbm.at[pl.ds(hvec[j], 1)],
                         out_h.at[pl.ds(base + gbase + j, 1)], hsem)
      for j in range(LANES):
        pltpu.async_copy(rel_hbm.at[pl.ds(rvec[j], 1)],
                         out_r.at[pl.ds(base + gbase + j, 1)], rsem)
      return _

    lax.fori_loop(0, NGROUP, fire, 0)

    def drain(i, _):
      pltpu.make_async_copy(ent_hbm.at[pl.ds(0, 1)],
                            out_h.at[pl.ds(base + i, 1)], hsem).wait()
      pltpu.make_async_copy(rel_hbm.at[pl.ds(0, 1)],
                            out_r.at[pl.ds(base + i, 1)], rsem).wait()
      return _

    lax.fori_loop(0, BPW, drain, 0)

  return lookup


_lookup = _make_sc_lookup()


@jax.jit
def kernel(heads, relations, entity_table, relation_table):
  heads_r = heads.astype(jnp.int32).reshape(NW, BPW)
  rels_r = relations.astype(jnp.int32).reshape(NW, BPW)
  return _lookup(heads_r, rels_r, entity_table, relation_table)


# speedup vs baseline: 2.2536x; 2.2536x over previous
import functools

import jax
import jax.numpy as jnp
from jax import lax
from jax.experimental import pallas as pl
from jax.experimental.pallas import tpu as pltpu
from jax.experimental.pallas import tpu_sc as plsc

NUM_ENTITIES = 1000000
NUM_RELATIONS = 1000
EMBED_DIM = 64
BATCH = 16384

NC = 2
NS = 16
NW = NC * NS
BPW = BATCH // NW
LANES = 16
NGROUP = BPW // LANES


def _make_sc_lookup():
  mesh = plsc.VectorSubcoreMesh(core_axis_name="c", subcore_axis_name="s")

  @functools.partial(
      pl.kernel,
      mesh=mesh,
      compiler_params=pltpu.CompilerParams(needs_layout_passes=False),
      out_type=(
          jax.ShapeDtypeStruct((BATCH, EMBED_DIM), jnp.float32),
          jax.ShapeDtypeStruct((BATCH, EMBED_DIM), jnp.float32),
      ),
      scratch_types=[
          pltpu.VMEM((BPW,), jnp.int32),
          pltpu.VMEM((BPW,), jnp.int32),
          pltpu.VMEM((BPW // 2, EMBED_DIM), jnp.float32),
          pltpu.VMEM((BPW // 2, EMBED_DIM), jnp.float32),
          pltpu.SemaphoreType.DMA,
          pltpu.SemaphoreType.DMA,
      ],
  )
  def lookup(heads_hbm, rels_hbm, ent_hbm, rel_hbm, out_h, out_r,
             hidx_v, ridx_v, hrows, rrows, hsem, rsem):
    wid = lax.axis_index("s") * NC + lax.axis_index("c")
    base = wid * BPW
    pltpu.sync_copy(heads_hbm.at[wid], hidx_v)
    pltpu.sync_copy(rels_hbm.at[wid], ridx_v)

    HALF = BPW // 2

    for w in range(2):
      wbase = w * HALF

      def fire(g, _, wbase=wbase):
        gbase = wbase + g * LANES
        hvec = hidx_v[pl.ds(gbase, LANES)]
        rvec = ridx_v[pl.ds(gbase, LANES)]
        for j in range(LANES):
          pltpu.async_copy(ent_hbm.at[pl.ds(hvec[j], 1)],
                           hrows.at[pl.ds(g * LANES + j, 1)], hsem)
        for j in range(LANES):
          pltpu.async_copy(rel_hbm.at[pl.ds(rvec[j], 1)],
                           rrows.at[pl.ds(g * LANES + j, 1)], rsem)
        return _

      lax.fori_loop(0, HALF // LANES, fire, 0)

      def drain(i, _):
        pltpu.make_async_copy(ent_hbm.at[pl.ds(0, 1)],
                              hrows.at[pl.ds(i, 1)], hsem).wait()
        pltpu.make_async_copy(rel_hbm.at[pl.ds(0, 1)],
                              rrows.at[pl.ds(i, 1)], rsem).wait()
        return _

      lax.fori_loop(0, HALF, drain, 0)
      pltpu.sync_copy(hrows, out_h.at[pl.ds(base + wbase, HALF)])
      pltpu.sync_copy(rrows, out_r.at[pl.ds(base + wbase, HALF)])

  return lookup


_lookup = _make_sc_lookup()


@jax.jit
def kernel(heads, relations, entity_table, relation_table):
  heads_r = heads.astype(jnp.int32).reshape(NW, BPW)
  rels_r = relations.astype(jnp.int32).reshape(NW, BPW)
  return _lookup(heads_r, rels_r, entity_table, relation_table)
